# baseline (speedup 1.0000x reference)
import jax
import jax.numpy as jnp
from jax import lax
from jax.experimental import pallas as pl
from jax.experimental.pallas import tpu as pltpu
from jax.experimental.pallas import tpu_sc as plsc

N = 50000
B = 1024
F = 256
TILE = 80
NT = N // TILE
NW = 32
ROUNDS = (NT + NW - 1) // NW
GROUPS = TILE // 16
KV = F // 16
MAXN = ROUNDS * TILE


def _umat_body(g_ref, w_ref, u_ref, umax_ref):
    x = jnp.dot(jnp.maximum(g_ref[...], 0.0), w_ref[...],
                preferred_element_type=jnp.float32)
    u_ref[...] = x
    umax_ref[...] = jnp.max(x, keepdims=True)


def _umat(g_feats, w1a):
    return pl.pallas_call(
        _umat_body,
        out_shape=[
            jax.ShapeDtypeStruct((B, 1), jnp.float32),
            jax.ShapeDtypeStruct((1, 1), jnp.float32),
        ],
    )(g_feats, w1a)


def _sc_body(node_hbm, seg_hbm, w_hbm, u_hbm, c_hbm, zp_hbm, zd_hbm,
             pooled_out, d_out, m_out,
             rows, segL, vL, eb, erows, tbuf, ub, wb, cb, mbuf,
             shp, shd, shm):
    cid = lax.axis_index("c")
    sid = lax.axis_index("s")
    wid = sid * 2 + cid

    pltpu.sync_copy(u_hbm, ub)
    pltpu.sync_copy(c_hbm, cb)
    pltpu.sync_copy(w_hbm, wb)
    zero16 = jnp.zeros((16,), jnp.float32)
    for i in range(TILE):
        erows[i, :] = zero16

    @pl.when(sid == 0)
    def _():
        pltpu.sync_copy(zp_hbm, shp)
        pltpu.sync_copy(zd_hbm, shd)

    cv = cb[...]
    b1s = cv[0]
    umax = cv[1]
    lanes = lax.iota(jnp.int32, 16)
    wbk = [wb[pl.ds(k * 16, 16)] for k in range(KV)]

    mbuf[...] = jnp.full((16,), -3.0e38, jnp.float32)

    def r1_body(r, carry):
        t = r * NW + wid

        @pl.when(t < NT)
        def _():
            base = t * TILE
            pltpu.sync_copy(node_hbm.at[pl.ds(base, TILE)], rows)
            pltpu.sync_copy(seg_hbm.at[pl.ds(base, TILE)],
                            segL.at[pl.ds(r * TILE, TILE)])
            for g in range(GROUPS):
                for j in range(16):
                    row0 = rows[g * 16 + j, pl.ds(0, 16)]
                    acc = row0 * wbk[0]
                    for k in range(1, KV):
                        acc = acc + rows[g * 16 + j, pl.ds(k * 16, 16)] * wbk[k]
                    plsc.store_scatter(tbuf, [lanes, jnp.full((16,), j, jnp.int32)],
                                       acc)
                vvec = tbuf[0, :]
                for i in range(1, 16):
                    vvec = vvec + tbuf[i, :]
                vL[pl.ds(r * TILE + g * 16, 16)] = vvec
                mbuf[...] = jnp.maximum(mbuf[...], vvec)

        return carry

    lax.fori_loop(0, ROUNDS, r1_body, 0)

    pltpu.sync_copy(mbuf, shm.at[sid])
    plsc.subcore_barrier()
    pltpu.sync_copy(shm, tbuf)
    allm = tbuf[0, :]
    for i in range(1, 16):
        allm = jnp.maximum(allm, tbuf[i, :])
    msc = jnp.maximum(lax.reduce_max(allm, (0,)) + umax + b1s, 0.0)

    @pl.when(sid == 0)
    def _():
        mbuf[...] = jnp.full((16,), msc, jnp.float32)
        pltpu.sync_copy(mbuf, m_out.at[cid])

    def r2_body(r, carry):
        t = r * NW + wid

        @pl.when(t < NT)
        def _():
            base = t * TILE
            pltpu.sync_copy(node_hbm.at[pl.ds(base, TILE)], rows)
            for g in range(GROUPS):
                sv = segL[pl.ds(r * TILE + g * 16, 16)]
                vv = vL[pl.ds(r * TILE + g * 16, 16)]
                uu = plsc.load_gather(ub, [sv])
                zz = vv + uu + b1s
                zz = jnp.where(zz > 0.0, zz, zz * 0.01)
                ee = jnp.exp(zz - msc)
                eb[pl.ds(g * 16, 16)] = ee
                plsc.store_scatter(erows, [g * 16 + lanes,
                                           jnp.zeros((16,), jnp.int32)], ee)

            def node_body(n, c2):
                es = plsc.load_gather(eb, [jnp.full((16,), n, jnp.int32)])
                for k in range(KV):
                    rows[n, pl.ds(k * 16, 16)] = rows[n, pl.ds(k * 16, 16)] * es
                return c2

            lax.fori_loop(0, TILE, node_body, 0)
            segb = segL.at[pl.ds(r * TILE, TILE)]
            pltpu.sync_copy(rows, shp.at[segb], add=True)
            pltpu.sync_copy(erows, shd.at[segb], add=True)

        return carry

    lax.fori_loop(0, ROUNDS, r2_body, 0)
    plsc.subcore_barrier()

    @pl.when(sid == 0)
    def _():
        pltpu.sync_copy(shp, pooled_out.at[cid])
        pltpu.sync_copy(shd, d_out.at[cid])


def _sc_pool(node_feats, seg, w1b, u, consts, zp, zd):
    mesh = plsc.VectorSubcoreMesh(core_axis_name="c", subcore_axis_name="s")
    kern = pl.kernel(
        _sc_body,
        mesh=mesh,
        compiler_params=pltpu.CompilerParams(
            needs_layout_passes=False, use_tc_tiling_on_sc=False),
        out_type=[
            jax.ShapeDtypeStruct((2, B, F), jnp.float32),
            jax.ShapeDtypeStruct((2, B, 16), jnp.float32),
            jax.ShapeDtypeStruct((2, 16), jnp.float32),
        ],
        scratch_types=[
            pltpu.VMEM((TILE, F), jnp.float32),
            pltpu.VMEM((MAXN,), jnp.int32),
            pltpu.VMEM((MAXN,), jnp.float32),
            pltpu.VMEM((TILE,), jnp.float32),
            pltpu.VMEM((TILE, 16), jnp.float32),
            pltpu.VMEM((16, 16), jnp.float32),
            pltpu.VMEM((B,), jnp.float32),
            pltpu.VMEM((F,), jnp.float32),
            pltpu.VMEM((16,), jnp.float32),
            pltpu.VMEM((16,), jnp.float32),
            pltpu.VMEM_SHARED((B, F), jnp.float32),
            pltpu.VMEM_SHARED((B, 16), jnp.float32),
            pltpu.VMEM_SHARED((16, 16), jnp.float32),
        ],
    )
    return kern(node_feats, seg, w1b, u, consts, zp, zd)


def _final_body(pp_ref, dp_ref, m_ref, g_ref, W2_ref, b2_ref, Wih_ref,
                Whh_ref, bih_ref, bhh_ref, out_ref):
    m_all = m_ref[...]
    m0 = m_all[0, 0]
    m1 = m_all[1, 0]
    mm = jnp.maximum(m0, m1)
    s0 = jnp.exp(m0 - mm)
    s1 = jnp.exp(m1 - mm)
    pooled = pp_ref[0] * s0 + pp_ref[1] * s1
    d = dp_ref[0, :, 0] * s0 + dp_ref[1, :, 0] * s1
    nonempty = d > 0.0
    inv = jnp.where(nonempty, 1.0 / jnp.where(nonempty, d, 1.0), 0.0)
    ctx_in = pooled * inv[:, None]
    dn = (((1,), (1,)), ((), ()))
    g_repr = lax.dot_general(ctx_in, W2_ref[...], dn,
                             preferred_element_type=jnp.float32)
    g_repr = g_repr + b2_ref[...][None, :] * nonempty[:, None].astype(jnp.float32)
    context = jnp.where(g_repr > 0.0, g_repr,
                        jnp.exp(jnp.minimum(g_repr, 0.0)) - 1.0)
    g = g_ref[...]
    gi = lax.dot_general(context, Wih_ref[...], dn,
                         preferred_element_type=jnp.float32) + bih_ref[...][None, :]
    gh = lax.dot_general(g, Whh_ref[...], dn,
                         preferred_element_type=jnp.float32) + bhh_ref[...][None, :]
    i_r, i_z, i_n = gi[:, :F], gi[:, F:2 * F], gi[:, 2 * F:]
    h_r, h_z, h_n = gh[:, :F], gh[:, F:2 * F], gh[:, 2 * F:]
    r = jax.nn.sigmoid(i_r + h_r)
    uu = jax.nn.sigmoid(i_z + h_z)
    n = jnp.tanh(i_n + r * h_n)
    out_ref[...] = (1.0 - uu) * n + uu * g


def _final(pooled_parts, d_parts, m_parts, g_feats, W2, b2, Wih, Whh, bih, bhh):
    return pl.pallas_call(
        _final_body,
        out_shape=jax.ShapeDtypeStruct((B, F), jnp.float32),
    )(pooled_parts, d_parts, m_parts, g_feats, W2, b2, Wih, Whh, bih, bhh)


def kernel(node_feats, g_feats, segment_ids, W1, b1, W2, b2, Wih, Whh, bih, bhh):
    w1a = W1[0, :F].reshape(F, 1)
    w1b = W1[0, F:]
    u2d, umax = _umat(g_feats, w1a)
    consts = jnp.concatenate(
        [b1, umax[0], jnp.zeros((14,), jnp.float32)]).astype(jnp.float32)
    zp = jnp.zeros((B, F), jnp.float32)
    zd = jnp.zeros((B, 16), jnp.float32)
    pooled_parts, d_parts, m_parts = _sc_pool(
        node_feats, segment_ids, w1b, u2d.reshape(B), consts, zp, zd)
    return _final(pooled_parts, d_parts, m_parts, g_feats, W2, b2, Wih, Whh,
                  bih, bhh)


# speedup vs baseline: 1.6571x; 1.6571x over previous
import functools

import jax
import jax.numpy as jnp
from jax import lax
from jax.experimental import pallas as pl
from jax.experimental.pallas import tpu as pltpu
from jax.experimental.pallas import tpu_sc as plsc

N = 50000
B = 1024
F = 256
TILE = 80
NT = N // TILE
NW = 32
ROUNDS = (NT + NW - 1) // NW
GROUPS = TILE // 16
KV = F // 16
ROW_BLK = 1000


def _matvec_body(node_ref, w_ref, v_ref, vmax_ref):
    x = jnp.dot(node_ref[...], w_ref[...], preferred_element_type=jnp.float32)
    v_ref[...] = x
    m = jnp.max(x, keepdims=True)
    pid = pl.program_id(0)

    @pl.when(pid == 0)
    def _():
        vmax_ref[...] = m

    @pl.when(pid != 0)
    def _():
        vmax_ref[...] = jnp.maximum(vmax_ref[...], m)


def _matvec(node_feats, w1b):
    return pl.pallas_call(
        _matvec_body,
        grid=(N // ROW_BLK,),
        in_specs=[
            pl.BlockSpec((ROW_BLK, F), lambda i: (i, 0)),
            pl.BlockSpec((F, 1), lambda i: (0, 0)),
        ],
        out_specs=[
            pl.BlockSpec((ROW_BLK, 1), lambda i: (i, 0)),
            pl.BlockSpec((1, 1), lambda i: (0, 0)),
        ],
        out_shape=[
            jax.ShapeDtypeStruct((N, 1), jnp.float32),
            jax.ShapeDtypeStruct((1, 1), jnp.float32),
        ],
    )(node_feats, w1b)


def _umat_body(g_ref, w_ref, u_ref, umax_ref):
    x = jnp.dot(jnp.maximum(g_ref[...], 0.0), w_ref[...],
                preferred_element_type=jnp.float32)
    u_ref[...] = x
    umax_ref[...] = jnp.max(x, keepdims=True)


def _umat(g_feats, w1a):
    return pl.pallas_call(
        _umat_body,
        out_shape=[
            jax.ShapeDtypeStruct((B, 1), jnp.float32),
            jax.ShapeDtypeStruct((1, 1), jnp.float32),
        ],
    )(g_feats, w1a)


def _sc_body(node_hbm, v_hbm, seg_hbm, u_hbm, c_hbm, zp_hbm, zd_hbm,
             pooled_out, d_out,
             rows_a, vb_a, segb_a, rows_b, vb_b, segb_b,
             eb, erows, ub, cb, shp, shd, sem_a, sem_b):
    cid = lax.axis_index("c")
    sid = lax.axis_index("s")
    wid = sid * 2 + cid

    pltpu.sync_copy(u_hbm, ub)
    pltpu.sync_copy(c_hbm, cb)
    zero16 = jnp.zeros((16,), jnp.float32)
    for i in range(TILE):
        erows[i, :] = zero16

    @pl.when(sid == 0)
    def _():
        pltpu.sync_copy(zp_hbm, shp)
        pltpu.sync_copy(zd_hbm, shd)

    plsc.subcore_barrier()

    cv = cb[...]
    b1s = cv[0]
    shift = cv[1]
    lanes = lax.iota(jnp.int32, 16)

    def start3(t, rows_x, vb_x, segb_x, sem_x):
        base = t * TILE
        pltpu.async_copy(node_hbm.at[pl.ds(base, TILE)], rows_x, sem_x)
        pltpu.async_copy(v_hbm.at[pl.ds(base, TILE)], vb_x, sem_x)
        pltpu.async_copy(seg_hbm.at[pl.ds(base, TILE)], segb_x, sem_x)

    def wait3(t, rows_x, vb_x, segb_x, sem_x):
        base = t * TILE
        pltpu.make_async_copy(node_hbm.at[pl.ds(base, TILE)], rows_x, sem_x).wait()
        pltpu.make_async_copy(v_hbm.at[pl.ds(base, TILE)], vb_x, sem_x).wait()
        pltpu.make_async_copy(seg_hbm.at[pl.ds(base, TILE)], segb_x, sem_x).wait()

    def process(rows_x, vb_x, segb_x):
        for g in range(GROUPS):
            sv = segb_x[pl.ds(g * 16, 16)]
            vv = vb_x[pl.ds(g * 16, 16)]
            uu = plsc.load_gather(ub, [sv])
            zz = vv + uu + b1s
            zz = jnp.where(zz > 0.0, zz, zz * 0.01)
            ee = jnp.exp(zz - shift)
            eb[pl.ds(g * 16, 16)] = ee
            plsc.store_scatter(erows, [g * 16 + lanes,
                                       jnp.zeros((16,), jnp.int32)], ee)

        def node_body(n, c2):
            es = plsc.load_gather(eb, [jnp.full((16,), n, jnp.int32)])
            for k in range(KV):
                rows_x[n, pl.ds(k * 16, 16)] = rows_x[n, pl.ds(k * 16, 16)] * es
            return c2

        lax.fori_loop(0, TILE, node_body, 0)
        pltpu.sync_copy(rows_x, shp.at[segb_x], add=True)
        pltpu.sync_copy(erows, shd.at[segb_x], add=True)

    t0 = wid

    @pl.when(t0 < NT)
    def _():
        start3(t0, rows_a, vb_a, segb_a, sem_a)

    def pair_body(i, carry):
        rr = i * 2
        t_a = rr * NW + wid
        t_b = (rr + 1) * NW + wid
        t_c = (rr + 2) * NW + wid

        @pl.when(t_b < NT)
        def _():
            start3(t_b, rows_b, vb_b, segb_b, sem_b)

        @pl.when(t_a < NT)
        def _():
            wait3(t_a, rows_a, vb_a, segb_a, sem_a)
            process(rows_a, vb_a, segb_a)

        @pl.when(t_c < NT)
        def _():
            start3(t_c, rows_a, vb_a, segb_a, sem_a)

        @pl.when(t_b < NT)
        def _():
            wait3(t_b, rows_b, vb_b, segb_b, sem_b)
            process(rows_b, vb_b, segb_b)

        return carry

    lax.fori_loop(0, ROUNDS // 2, pair_body, 0)
    plsc.subcore_barrier()

    @pl.when(sid == 0)
    def _():
        pltpu.sync_copy(shp, pooled_out.at[cid])
        pltpu.sync_copy(shd, d_out.at[cid])


def _sc_pool(node_feats, v, seg, u, consts, zp, zd):
    mesh = plsc.VectorSubcoreMesh(core_axis_name="c", subcore_axis_name="s")
    kern = pl.kernel(
        _sc_body,
        mesh=mesh,
        compiler_params=pltpu.CompilerParams(
            needs_layout_passes=False, use_tc_tiling_on_sc=False),
        out_type=[
            jax.ShapeDtypeStruct((2, B, F), jnp.float32),
            jax.ShapeDtypeStruct((2, B, 16), jnp.float32),
        ],
        scratch_types=[
            pltpu.VMEM((TILE, F), jnp.float32),
            pltpu.VMEM((TILE,), jnp.float32),
            pltpu.VMEM((TILE,), jnp.int32),
            pltpu.VMEM((TILE, F), jnp.float32),
            pltpu.VMEM((TILE,), jnp.float32),
            pltpu.VMEM((TILE,), jnp.int32),
            pltpu.VMEM((TILE,), jnp.float32),
            pltpu.VMEM((TILE, 16), jnp.float32),
            pltpu.VMEM((B,), jnp.float32),
            pltpu.VMEM((16,), jnp.float32),
            pltpu.VMEM_SHARED((B, F), jnp.float32),
            pltpu.VMEM_SHARED((B, 16), jnp.float32),
            pltpu.SemaphoreType.DMA,
            pltpu.SemaphoreType.DMA,
        ],
    )
    return kern(node_feats, v, seg, u, consts, zp, zd)


def _final_body(pp_ref, dp_ref, g_ref, W2_ref, b2_ref, Wih_ref, Whh_ref,
                bih_ref, bhh_ref, out_ref):
    pooled = pp_ref[0] + pp_ref[1]
    d = dp_ref[0, :, 0] + dp_ref[1, :, 0]
    nonempty = d > 0.0
    inv = jnp.where(nonempty, 1.0 / jnp.where(nonempty, d, 1.0), 0.0)
    ctx_in = pooled * inv[:, None]
    dn = (((1,), (1,)), ((), ()))
    g_repr = lax.dot_general(ctx_in, W2_ref[...], dn,
                             preferred_element_type=jnp.float32)
    g_repr = g_repr + b2_ref[...][None, :] * nonempty[:, None].astype(jnp.float32)
    context = jnp.where(g_repr > 0.0, g_repr,
                        jnp.exp(jnp.minimum(g_repr, 0.0)) - 1.0)
    g = g_ref[...]
    gi = lax.dot_general(context, Wih_ref[...], dn,
                         preferred_element_type=jnp.float32) + bih_ref[...][None, :]
    gh = lax.dot_general(g, Whh_ref[...], dn,
                         preferred_element_type=jnp.float32) + bhh_ref[...][None, :]
    i_r, i_z, i_n = gi[:, :F], gi[:, F:2 * F], gi[:, 2 * F:]
    h_r, h_z, h_n = gh[:, :F], gh[:, F:2 * F], gh[:, 2 * F:]
    r = jax.nn.sigmoid(i_r + h_r)
    uu = jax.nn.sigmoid(i_z + h_z)
    n = jnp.tanh(i_n + r * h_n)
    out_ref[...] = (1.0 - uu) * n + uu * g


def _final(pooled_parts, d_parts, g_feats, W2, b2, Wih, Whh, bih, bhh):
    return pl.pallas_call(
        _final_body,
        out_shape=jax.ShapeDtypeStruct((B, F), jnp.float32),
    )(pooled_parts, d_parts, g_feats, W2, b2, Wih, Whh, bih, bhh)


def kernel(node_feats, g_feats, segment_ids, W1, b1, W2, b2, Wih, Whh, bih, bhh):
    w1a = W1[0, :F].reshape(F, 1)
    w1b = W1[0, F:].reshape(F, 1)
    v2d, vmax = _matvec(node_feats, w1b)
    u2d, umax = _umat(g_feats, w1a)
    M = jnp.maximum(vmax[0, 0] + umax[0, 0] + b1[0], 0.0)
    consts = jnp.concatenate(
        [b1, M[None], jnp.zeros((14,), jnp.float32)]).astype(jnp.float32)
    zp = jnp.zeros((B, F), jnp.float32)
    zd = jnp.zeros((B, 16), jnp.float32)
    pooled_parts, d_parts = _sc_pool(
        node_feats, v2d.reshape(N), segment_ids, u2d.reshape(B), consts, zp, zd)
    return _final(pooled_parts, d_parts, g_feats, W2, b2, Wih, Whh, bih, bhh)
